# baseline (speedup 1.0000x reference)
import jax
import jax.numpy as jnp
from jax import lax
from jax.experimental import pallas as pl
from jax.experimental.pallas import tpu as pltpu
from jax.experimental.pallas import tpu_sc as plsc

_B, _S, _D = 4, 4096, 1024
_T = 4
_PD = _D // _T
_N = _B * _S
_NW = 32
_TOK = _N // _NW
_CHUNK = 16
_NCH = _TOK // _CHUNK


def _sc_body(x_ref, idx_ref, pe0_ref, pe1_ref, pe2_ref, pe3_ref, out_ref,
             idx_v, r0, r1, x0, x1, sg0, sg1, sx0, sx1, so0, so1):
    c = lax.axis_index("c")
    s = lax.axis_index("s")
    wid = s * 2 + c
    tok_base = wid * _TOK
    tables = (pe0_ref, pe1_ref, pe2_ref, pe3_ref)
    rows = (r0, r1)
    xs = (x0, x1)
    sgs = (sg0, sg1)
    sxs = (sx0, sx1)
    sos = (so0, so1)

    for ti in range(_T):
        pltpu.sync_copy(idx_ref.at[ti, pl.ds(tok_base, _TOK)], idx_v.at[ti])

    def start_in(ci, b):
        for ti in range(_T):
            pltpu.async_copy(
                tables[ti].at[idx_v.at[ti, pl.ds(ci * _CHUNK, _CHUNK)]],
                rows[b].at[ti], sgs[b])
        pltpu.async_copy(
            x_ref.at[pl.ds(tok_base + ci * _CHUNK, _CHUNK)], xs[b], sxs[b])

    def finish(ci, b):
        pltpu.make_async_copy(x_ref.at[pl.ds(0, _T * _CHUNK), pl.ds(0, _PD)],
                              rows[b], sgs[b]).wait()
        pltpu.make_async_copy(x_ref.at[pl.ds(0, _CHUNK)], xs[b], sxs[b]).wait()

        def add_row(r, carry):
            for ti in range(_T):
                for j in range(_PD // 16):
                    plsc.addupdate(
                        xs[b].at[r, pl.ds(ti * _PD + j * 16, 16)],
                        rows[b][ti, r, pl.ds(j * 16, 16)])
            return carry

        lax.fori_loop(0, _CHUNK, add_row, 0)
        pltpu.async_copy(
            xs[b], out_ref.at[pl.ds(tok_base + ci * _CHUNK, _CHUNK)], sos[b])

    def drain_out(b):
        pltpu.make_async_copy(
            xs[b], out_ref.at[pl.ds(0, _CHUNK)], sos[b]).wait()

    start_in(0, 0)

    def ring(g2, carry):
        g = g2 * 2
        @pl.when(g >= 1)
        def _():
            drain_out(1)
        start_in(g + 1, 1)
        finish(g, 0)
        @pl.when(g + 2 < _NCH)
        def _():
            drain_out(0)
            start_in(g + 2, 0)
        finish(g + 1, 1)
        return carry

    lax.fori_loop(0, _NCH // 2, ring, 0)
    drain_out(0)
    drain_out(1)


@jax.jit
def kernel(x, pe0, pe1, pe2, pe3, indexes):
    xf = x.reshape(_N, _D)
    idx = indexes.reshape(_N, _T).T
    mesh = plsc.VectorSubcoreMesh(core_axis_name="c", subcore_axis_name="s")
    out = pl.kernel(
        _sc_body,
        out_type=jax.ShapeDtypeStruct((_N, _D), jnp.float32),
        mesh=mesh,
        scratch_types=[
            pltpu.VMEM((_T, _TOK), jnp.int32),
            pltpu.VMEM((_T, _CHUNK, _PD), jnp.float32),
            pltpu.VMEM((_T, _CHUNK, _PD), jnp.float32),
            pltpu.VMEM((_CHUNK, _D), jnp.float32),
            pltpu.VMEM((_CHUNK, _D), jnp.float32),
            pltpu.SemaphoreType.DMA,
            pltpu.SemaphoreType.DMA,
            pltpu.SemaphoreType.DMA,
            pltpu.SemaphoreType.DMA,
            pltpu.SemaphoreType.DMA,
            pltpu.SemaphoreType.DMA,
        ],
    )(xf, idx, pe0, pe1, pe2, pe3)
    return out.reshape(_B, _S, _D)


# speedup vs baseline: 1.7392x; 1.7392x over previous
import jax
import jax.numpy as jnp
from jax import lax
from jax.experimental import pallas as pl
from jax.experimental.pallas import tpu as pltpu
from jax.experimental.pallas import tpu_sc as plsc

_B, _S, _D = 4, 4096, 1024
_T = 4
_PD = _D // _T
_N = _B * _S
_NW = 32
_TPT = _NW // _T
_TOK = _N // _TPT
_CHUNK = 64
_NCH = _TOK // _CHUNK
_DEPTH = 3


def _sc_body(x_ref, idx_ref, pe0_ref, pe1_ref, pe2_ref, pe3_ref, out_ref,
             idx_v, r0, r1, r2, x0, x1, x2,
             sg0, sg1, sg2, sx0, sx1, sx2, so0, so1, so2):
    c = lax.axis_index("c")
    s = lax.axis_index("s")
    wid = s * 2 + c
    t = wid // _TPT
    tok_base = (wid % _TPT) * _TOK
    col = t * _PD
    tables = (pe0_ref, pe1_ref, pe2_ref, pe3_ref)
    rows = (r0, r1, r2)
    xs = (x0, x1, x2)
    sgs = (sg0, sg1, sg2)
    sxs = (sx0, sx1, sx2)
    sos = (so0, so1, so2)

    pltpu.sync_copy(idx_ref.at[t, pl.ds(tok_base, _TOK)], idx_v)

    def start_in(ci, b):
        for ti in range(_T):
            @pl.when(t == ti)
            def _():
                pltpu.async_copy(
                    tables[ti].at[idx_v.at[pl.ds(ci * _CHUNK, _CHUNK)]],
                    rows[b], sgs[b])
        pltpu.async_copy(
            x_ref.at[pl.ds(tok_base + ci * _CHUNK, _CHUNK), pl.ds(col, _PD)],
            xs[b], sxs[b])

    def finish(ci, b):
        pltpu.make_async_copy(tables[0].at[idx_v.at[pl.ds(0, _CHUNK)]],
                              rows[b], sgs[b]).wait()
        pltpu.make_async_copy(x_ref.at[pl.ds(0, _CHUNK), pl.ds(0, _PD)],
                              xs[b], sxs[b]).wait()

        def add_row(r, carry):
            for j in range(_PD // 16):
                plsc.addupdate(xs[b].at[r, pl.ds(j * 16, 16)],
                               rows[b][r, pl.ds(j * 16, 16)])
            return carry

        lax.fori_loop(0, _CHUNK, add_row, 0)
        pltpu.async_copy(
            xs[b],
            out_ref.at[pl.ds(tok_base + ci * _CHUNK, _CHUNK), pl.ds(col, _PD)],
            sos[b])

    def drain_out(b):
        pltpu.make_async_copy(
            xs[b], out_ref.at[pl.ds(0, _CHUNK), pl.ds(0, _PD)], sos[b]).wait()

    start_in(0, 0)
    start_in(1, 1)

    def step(c0, b):
        @pl.when(jnp.logical_and(c0 >= 1, c0 + 2 < _NCH))
        def _():
            drain_out((b + 2) % _DEPTH)
        @pl.when(c0 + 2 < _NCH)
        def _():
            start_in(c0 + 2, (b + 2) % _DEPTH)
        finish(c0, b)

    def ring(g3, carry):
        g = g3 * _DEPTH
        for k in range(_DEPTH):
            step(g + k, k)
        return carry

    lax.fori_loop(0, _NCH // _DEPTH, ring, 0)
    for k in range(_NCH - _NCH % _DEPTH, _NCH):
        step(k, k % _DEPTH)
    for k in range(_NCH - _DEPTH, _NCH):
        drain_out(k % _DEPTH)


@jax.jit
def kernel(x, pe0, pe1, pe2, pe3, indexes):
    xf = x.reshape(_N, _D)
    idx = indexes.reshape(_N, _T).T
    mesh = plsc.VectorSubcoreMesh(core_axis_name="c", subcore_axis_name="s")
    out = pl.kernel(
        _sc_body,
        out_type=jax.ShapeDtypeStruct((_N, _D), jnp.float32),
        mesh=mesh,
        scratch_types=[
            pltpu.VMEM((_TOK,), jnp.int32),
            pltpu.VMEM((_CHUNK, _PD), jnp.float32),
            pltpu.VMEM((_CHUNK, _PD), jnp.float32),
            pltpu.VMEM((_CHUNK, _PD), jnp.float32),
            pltpu.VMEM((_CHUNK, _PD), jnp.float32),
            pltpu.VMEM((_CHUNK, _PD), jnp.float32),
            pltpu.VMEM((_CHUNK, _PD), jnp.float32),
            pltpu.SemaphoreType.DMA,
            pltpu.SemaphoreType.DMA,
            pltpu.SemaphoreType.DMA,
            pltpu.SemaphoreType.DMA,
            pltpu.SemaphoreType.DMA,
            pltpu.SemaphoreType.DMA,
            pltpu.SemaphoreType.DMA,
            pltpu.SemaphoreType.DMA,
            pltpu.SemaphoreType.DMA,
        ],
    )(xf, idx, pe0, pe1, pe2, pe3)
    return out.reshape(_B, _S, _D)
